# baseline (speedup 1.0000x reference)
import jax
import jax.numpy as jnp
from jax import lax
from jax.experimental import pallas as pl
from jax.experimental.pallas import tpu as pltpu
from jax.experimental.pallas import tpu_sc as plsc

N = 10000
E = 320000
F_IN = 128
HID = 16
NCLS = 7

NC = 2
NS = 16
NW = NC * NS
EPW = E // NW
NCH = 25
CHUNK = EPW // NCH
NSLOT = 12
EPT = E // NS
NHALF = N // NC
NPT = 320
NPT_LAST = NHALF - (NS - 1) * NPT


def _sc_mesh():
    return plsc.VectorSubcoreMesh(core_axis_name="c", subcore_axis_name="s")


def _newton_rsqrt(x):
    i = plsc.bitcast(x, jnp.int32)
    i = 0x5F3759DF - lax.shift_right_logical(i, 1)
    y = plsc.bitcast(i, jnp.float32)
    for _ in range(3):
        y = y * (1.5 - 0.5 * x * y * y)
    return y


def _deg_body(ei_hbm, z_hbm, out_hbm, dst_v, hist_v, acc_v, slot_v, dinv16_v,
              sp_hist):
    cid = lax.axis_index("c")
    sid = lax.axis_index("s")
    pltpu.sync_copy(ei_hbm.at[pl.ds(E + sid * EPT, EPT)], dst_v)
    pltpu.sync_copy(z_hbm, hist_v)
    ones = jnp.full((16,), 1.0, dtype=jnp.float32)

    def hbody(i, carry):
        d = dst_v[pl.ds(i * 16, 16)]
        plsc.addupdate_scatter(hist_v, [d], ones)
        return carry

    lax.fori_loop(0, EPT // 16, hbody, 0)

    pltpu.sync_copy(hist_v, sp_hist.at[sid])
    plsc.subcore_barrier()

    def finish(nn, start):
        nv = (nn + 15) // 16
        pltpu.sync_copy(sp_hist.at[:, pl.ds(start, nn)],
                        slot_v.at[:, pl.ds(0, nn)])

        def rsq_body(i, carry):
            d = jnp.full((16,), 0.0, jnp.float32)
            for k in range(NS):
                d = d + slot_v[k, pl.ds(i * 16, 16)]
            acc_v[pl.ds(i * 16, 16)] = _newton_rsqrt(1.0 + d)
            return carry

        lax.fori_loop(0, nv, rsq_body, 0)

        def exp_body(v, carry):
            idx = jnp.full((16,), 0, jnp.int32) + v
            val = plsc.load_gather(acc_v, [idx])
            dinv16_v[pl.ds(v * 16, 16)] = val
            return carry

        lax.fori_loop(0, nn, exp_body, 0)
        pltpu.sync_copy(dinv16_v.at[pl.ds(0, nn * 16)],
                        out_hbm.at[pl.ds(start * 16, nn * 16)])

    @pl.when(sid < NS - 1)
    def _():
        finish(NPT, cid * NHALF + sid * NPT)

    @pl.when(sid == NS - 1)
    def _():
        finish(NPT_LAST, cid * NHALF + (NS - 1) * NPT)


def _deg_call(ei_flat, zerosN):
    k = pl.kernel(
        _deg_body,
        out_type=jax.ShapeDtypeStruct((N * 16,), jnp.float32),
        mesh=_sc_mesh(),
        scratch_types=[
            pltpu.VMEM((EPT,), jnp.int32),
            pltpu.VMEM((N,), jnp.float32),
            pltpu.VMEM((NPT,), jnp.float32),
            pltpu.VMEM((NS, NPT), jnp.float32),
            pltpu.VMEM((NPT * 16,), jnp.float32),
            pltpu.VMEM_SHARED((NS, N), jnp.float32),
        ],
        compiler_params=pltpu.CompilerParams(
            needs_layout_passes=False, use_tc_tiling_on_sc=False),
    )
    return k(ei_flat, zerosN)


def _agg_body(g_hbm, ei_hbm, z_hbm, out_hbm, src_v, dst_v, rows_v,
              acc_sh, gsem, ssem, isem, zsem):
    cid = lax.axis_index("c")
    sid = lax.axis_index("s")
    wid = sid * NC + cid

    @pl.when(sid == 0)
    def _():
        pltpu.async_copy(z_hbm, acc_sh, zsem)

    def stage(j):
        return (
            pltpu.make_async_copy(
                ei_hbm.at[pl.ds(wid * EPW + j * CHUNK, CHUNK)],
                src_v.at[j], isem),
            pltpu.make_async_copy(
                ei_hbm.at[pl.ds(E + wid * EPW + j * CHUNK, CHUNK)],
                dst_v.at[j], isem),
        )

    for j in range(NCH):
        for c in stage(j):
            c.start()
    for j in range(NCH):
        for c in stage(j):
            c.wait()

    @pl.when(sid == 0)
    def _():
        pltpu.make_async_copy(z_hbm, acc_sh, zsem).wait()

    plsc.subcore_barrier()

    def start_gather(j, slot):
        pltpu.async_copy(g_hbm.at[src_v.at[j]], rows_v.at[slot], gsem.at[slot])

    def wait_gather(j, slot):
        pltpu.make_async_copy(g_hbm.at[src_v.at[j]], rows_v.at[slot],
                              gsem.at[slot]).wait()

    def start_scatter(j, slot):
        pltpu.async_copy(rows_v.at[slot], acc_sh.at[dst_v.at[j]],
                         ssem.at[slot], add=True)

    def wait_scatter(j, slot):
        pltpu.make_async_copy(rows_v.at[slot], acc_sh.at[dst_v.at[j]],
                              ssem.at[slot]).wait()

    for j in range(NSLOT - 1):
        start_gather(j, j)

    def chunk(j, carry):
        slot = lax.rem(j, NSLOT)

        @pl.when(j >= 1)
        def _():
            wait_scatter(j - 1, lax.rem(j + NSLOT - 1, NSLOT))

        wait_gather(j, slot)
        start_scatter(j, slot)

        @pl.when(j + NSLOT - 1 < NCH)
        def _():
            start_gather(j + NSLOT - 1, lax.rem(j + NSLOT - 1, NSLOT))

        return carry

    lax.fori_loop(0, NCH, chunk, 0)
    wait_scatter(NCH - 1, lax.rem(NCH - 1, NSLOT))
    plsc.subcore_barrier()

    @pl.when(sid == 0)
    def _():
        pltpu.sync_copy(acc_sh, out_hbm.at[cid])


def _agg_call(g, ei_flat, zeros):
    k = pl.kernel(
        _agg_body,
        out_type=jax.ShapeDtypeStruct((NC, N, 16), jnp.float32),
        mesh=_sc_mesh(),
        scratch_types=[
            pltpu.VMEM((NCH, CHUNK), jnp.int32),
            pltpu.VMEM((NCH, CHUNK), jnp.int32),
            pltpu.VMEM((NSLOT, CHUNK, 16), jnp.float32),
            pltpu.VMEM_SHARED((N, 16), jnp.float32),
            pltpu.SemaphoreType.DMA((NSLOT,)),
            pltpu.SemaphoreType.DMA((NSLOT,)),
            pltpu.SemaphoreType.DMA,
            pltpu.SemaphoreType.DMA,
        ],
        compiler_params=pltpu.CompilerParams(
            needs_layout_passes=False, use_tc_tiling_on_sc=False),
    )
    return k(g, ei_flat, zeros)


def _tc1a_body(xr_ref, w1bd_ref, h_ref):
    h_ref[...] = jnp.dot(xr_ref[...], w1bd_ref[...],
                         preferred_element_type=jnp.float32)


def _tc1a_call(x_r, W1bd):
    return pl.pallas_call(
        _tc1a_body,
        out_shape=jax.ShapeDtypeStruct((N // 8, 128), jnp.float32),
    )(x_r, W1bd)


def _tc1b_body(h_ref, dinv_ref, g1_ref):
    g1_ref[...] = h_ref[...] * dinv_ref[...]


def _tc1b_call(h128, dinv128):
    return pl.pallas_call(
        _tc1b_body,
        out_shape=jax.ShapeDtypeStruct((N // 8, 128), jnp.float32),
    )(h128, dinv128)


def _tc2_body(a_ref, g1_ref, dinv_ref, b1_ref, w2bd_ref, g2_ref):
    s = a_ref[0:N // 8, :] + a_ref[N // 8:, :] + g1_ref[...]
    dinv = dinv_ref[...]
    out1 = jnp.maximum(dinv * s + b1_ref[...], 0.0)
    h2 = jnp.dot(out1, w2bd_ref[...], preferred_element_type=jnp.float32)
    g2_ref[...] = h2 * dinv


def _tc2_call(a128, g1_128, dinv128, b1_128, W2bd):
    return pl.pallas_call(
        _tc2_body,
        out_shape=jax.ShapeDtypeStruct((N // 8, 128), jnp.float32),
    )(a128, g1_128, dinv128, b1_128, W2bd)


def _tc3_body(a_ref, g2_ref, dinv_ref, b2_ref, o_ref):
    s = a_ref[0:N // 8, :] + a_ref[N // 8:, :] + g2_ref[...]
    o_ref[...] = dinv_ref[...] * s + b2_ref[...]


def _tc3_call(a128, g2_128, dinv128, b2_128):
    return pl.pallas_call(
        _tc3_body,
        out_shape=jax.ShapeDtypeStruct((N // 8, 128), jnp.float32),
    )(a128, g2_128, dinv128, b2_128)


@jax.jit
def kernel(x, edge_index, W1, b1, W2, b2):
    f32 = jnp.float32
    ei_flat = edge_index.reshape(2 * E)
    zerosN = jnp.zeros((N,), f32)
    zeros = jnp.zeros((N, 16), f32)
    eye8 = jnp.eye(8, dtype=f32)
    W1bd = jnp.kron(eye8, W1)
    W2p = jnp.zeros((HID, 16), f32).at[:, :NCLS].set(W2)
    W2bd = jnp.kron(eye8, W2p)
    b1_128 = jnp.tile(b1, 8)[None, :]
    b2p = jnp.zeros((16,), f32).at[:NCLS].set(b2)
    b2_128 = jnp.tile(b2p, 8)[None, :]
    x_r = x.reshape(N // 8, 8 * F_IN)

    h1_128 = _tc1a_call(x_r, W1bd)
    dinv16 = _deg_call(ei_flat, zerosN)
    dinv128 = dinv16.reshape(N // 8, 128)

    g1_128 = _tc1b_call(h1_128, dinv128)
    g1 = g1_128.reshape(N, 16)

    agg1 = _agg_call(g1, ei_flat, zeros)
    a1_128 = agg1.reshape(2 * (N // 8), 128)
    g2_128 = _tc2_call(a1_128, g1_128, dinv128, b1_128, W2bd)
    g2 = g2_128.reshape(N, 16)

    agg2 = _agg_call(g2, ei_flat, zeros)
    a2_128 = agg2.reshape(2 * (N // 8), 128)
    o_128 = _tc3_call(a2_128, g2_128, dinv128, b2_128)
    return o_128.reshape(N, 16)[:, :NCLS]


# speedup vs baseline: 1.0059x; 1.0059x over previous
import jax
import jax.numpy as jnp
from jax import lax
from jax.experimental import pallas as pl
from jax.experimental.pallas import tpu as pltpu
from jax.experimental.pallas import tpu_sc as plsc

N = 10000
E = 320000
F_IN = 128
HID = 16
NCLS = 7

NC = 2
NS = 16
NW = NC * NS
EPW = E // NW
NCH = 25
CHUNK = EPW // NCH
NSLOT = 12
EPT = E // NS
NHALF = N // NC
NPT = 320
NPT_LAST = NHALF - (NS - 1) * NPT


def _sc_mesh():
    return plsc.VectorSubcoreMesh(core_axis_name="c", subcore_axis_name="s")


def _newton_rsqrt(x):
    i = plsc.bitcast(x, jnp.int32)
    i = 0x5F3759DF - lax.shift_right_logical(i, 1)
    y = plsc.bitcast(i, jnp.float32)
    for _ in range(3):
        y = y * (1.5 - 0.5 * x * y * y)
    return y


def _deg_body(ei_hbm, z_hbm, out_hbm, dst_v, hist_v, acc_v, slot_v, dinv16_v,
              sp_hist):
    cid = lax.axis_index("c")
    sid = lax.axis_index("s")
    pltpu.sync_copy(ei_hbm.at[pl.ds(E + sid * EPT, EPT)], dst_v)
    pltpu.sync_copy(z_hbm, hist_v)
    ones = jnp.full((16,), 1.0, dtype=jnp.float32)

    def hbody(i, carry):
        d = dst_v[pl.ds(i * 16, 16)]
        plsc.addupdate_scatter(hist_v, [d], ones)
        return carry

    lax.fori_loop(0, EPT // 16, hbody, 0)

    pltpu.sync_copy(hist_v, sp_hist.at[sid])
    plsc.subcore_barrier()

    def finish(nn, start):
        nv = (nn + 15) // 16
        pltpu.sync_copy(sp_hist.at[:, pl.ds(start, nn)],
                        slot_v.at[:, pl.ds(0, nn)])

        def rsq_body(i, carry):
            d = jnp.full((16,), 0.0, jnp.float32)
            for k in range(NS):
                d = d + slot_v[k, pl.ds(i * 16, 16)]
            acc_v[pl.ds(i * 16, 16)] = _newton_rsqrt(1.0 + d)
            return carry

        lax.fori_loop(0, nv, rsq_body, 0)

        def exp_body(v, carry):
            idx = jnp.full((16,), 0, jnp.int32) + v
            val = plsc.load_gather(acc_v, [idx])
            dinv16_v[pl.ds(v * 16, 16)] = val
            return carry

        lax.fori_loop(0, nn, exp_body, 0)
        pltpu.sync_copy(dinv16_v.at[pl.ds(0, nn * 16)],
                        out_hbm.at[pl.ds(start * 16, nn * 16)])

    @pl.when(sid < NS - 1)
    def _():
        finish(NPT, cid * NHALF + sid * NPT)

    @pl.when(sid == NS - 1)
    def _():
        finish(NPT_LAST, cid * NHALF + (NS - 1) * NPT)


def _deg_call(ei_flat, zerosN):
    k = pl.kernel(
        _deg_body,
        out_type=jax.ShapeDtypeStruct((N * 16,), jnp.float32),
        mesh=_sc_mesh(),
        scratch_types=[
            pltpu.VMEM((EPT,), jnp.int32),
            pltpu.VMEM((N,), jnp.float32),
            pltpu.VMEM((NPT,), jnp.float32),
            pltpu.VMEM((NS, NPT), jnp.float32),
            pltpu.VMEM((NPT * 16,), jnp.float32),
            pltpu.VMEM_SHARED((NS, N), jnp.float32),
        ],
        compiler_params=pltpu.CompilerParams(
            needs_layout_passes=False, use_tc_tiling_on_sc=False),
    )
    return k(ei_flat, zerosN)


def _agg_body(g_hbm, ei_hbm, z_hbm, out_hbm, src_v, dst_v, rows_v,
              acc_sh, g_sh, gsem, ssem, isem, zsem, gssem):
    cid = lax.axis_index("c")
    sid = lax.axis_index("s")
    wid = sid * NC + cid

    @pl.when(sid == 0)
    def _():
        pltpu.async_copy(z_hbm, acc_sh, zsem)
        pltpu.async_copy(g_hbm, g_sh, gssem)

    def stage(j):
        return (
            pltpu.make_async_copy(
                ei_hbm.at[pl.ds(wid * EPW + j * CHUNK, CHUNK)],
                src_v.at[j], isem),
            pltpu.make_async_copy(
                ei_hbm.at[pl.ds(E + wid * EPW + j * CHUNK, CHUNK)],
                dst_v.at[j], isem),
        )

    for j in range(NCH):
        for c in stage(j):
            c.start()
    for j in range(NCH):
        for c in stage(j):
            c.wait()

    @pl.when(sid == 0)
    def _():
        pltpu.make_async_copy(z_hbm, acc_sh, zsem).wait()
        pltpu.make_async_copy(g_hbm, g_sh, gssem).wait()

    plsc.subcore_barrier()

    def start_gather(j, slot):
        pltpu.async_copy(g_sh.at[src_v.at[j]], rows_v.at[slot], gsem.at[slot])

    def wait_gather(j, slot):
        pltpu.make_async_copy(g_sh.at[src_v.at[j]], rows_v.at[slot],
                              gsem.at[slot]).wait()

    def start_scatter(j, slot):
        pltpu.async_copy(rows_v.at[slot], acc_sh.at[dst_v.at[j]],
                         ssem.at[slot], add=True)

    def wait_scatter(j, slot):
        pltpu.make_async_copy(rows_v.at[slot], acc_sh.at[dst_v.at[j]],
                              ssem.at[slot]).wait()

    for j in range(NSLOT - 1):
        start_gather(j, j)

    def chunk(j, carry):
        slot = lax.rem(j, NSLOT)

        @pl.when(j >= 1)
        def _():
            wait_scatter(j - 1, lax.rem(j + NSLOT - 1, NSLOT))

        wait_gather(j, slot)
        start_scatter(j, slot)

        @pl.when(j + NSLOT - 1 < NCH)
        def _():
            start_gather(j + NSLOT - 1, lax.rem(j + NSLOT - 1, NSLOT))

        return carry

    lax.fori_loop(0, NCH, chunk, 0)
    wait_scatter(NCH - 1, lax.rem(NCH - 1, NSLOT))
    plsc.subcore_barrier()

    @pl.when(sid == 0)
    def _():
        pltpu.sync_copy(acc_sh, out_hbm.at[cid])


def _agg_call(g, ei_flat, zeros):
    k = pl.kernel(
        _agg_body,
        out_type=jax.ShapeDtypeStruct((NC, N, 16), jnp.float32),
        mesh=_sc_mesh(),
        scratch_types=[
            pltpu.VMEM((NCH, CHUNK), jnp.int32),
            pltpu.VMEM((NCH, CHUNK), jnp.int32),
            pltpu.VMEM((NSLOT, CHUNK, 16), jnp.float32),
            pltpu.VMEM_SHARED((N, 16), jnp.float32),
            pltpu.VMEM_SHARED((N, 16), jnp.float32),
            pltpu.SemaphoreType.DMA((NSLOT,)),
            pltpu.SemaphoreType.DMA((NSLOT,)),
            pltpu.SemaphoreType.DMA,
            pltpu.SemaphoreType.DMA,
            pltpu.SemaphoreType.DMA,
        ],
        compiler_params=pltpu.CompilerParams(
            needs_layout_passes=False, use_tc_tiling_on_sc=False),
    )
    return k(g, ei_flat, zeros)


def _tc1a_body(xr_ref, w1bd_ref, h_ref):
    h_ref[...] = jnp.dot(xr_ref[...], w1bd_ref[...],
                         preferred_element_type=jnp.float32)


def _tc1a_call(x_r, W1bd):
    return pl.pallas_call(
        _tc1a_body,
        out_shape=jax.ShapeDtypeStruct((N // 8, 128), jnp.float32),
    )(x_r, W1bd)


def _tc1b_body(h_ref, dinv_ref, g1_ref):
    g1_ref[...] = h_ref[...] * dinv_ref[...]


def _tc1b_call(h128, dinv128):
    return pl.pallas_call(
        _tc1b_body,
        out_shape=jax.ShapeDtypeStruct((N // 8, 128), jnp.float32),
    )(h128, dinv128)


def _tc2_body(a_ref, g1_ref, dinv_ref, b1_ref, w2bd_ref, g2_ref):
    s = a_ref[0:N // 8, :] + a_ref[N // 8:, :] + g1_ref[...]
    dinv = dinv_ref[...]
    out1 = jnp.maximum(dinv * s + b1_ref[...], 0.0)
    h2 = jnp.dot(out1, w2bd_ref[...], preferred_element_type=jnp.float32)
    g2_ref[...] = h2 * dinv


def _tc2_call(a128, g1_128, dinv128, b1_128, W2bd):
    return pl.pallas_call(
        _tc2_body,
        out_shape=jax.ShapeDtypeStruct((N // 8, 128), jnp.float32),
    )(a128, g1_128, dinv128, b1_128, W2bd)


def _tc3_body(a_ref, g2_ref, dinv_ref, b2_ref, o_ref):
    s = a_ref[0:N // 8, :] + a_ref[N // 8:, :] + g2_ref[...]
    o_ref[...] = dinv_ref[...] * s + b2_ref[...]


def _tc3_call(a128, g2_128, dinv128, b2_128):
    return pl.pallas_call(
        _tc3_body,
        out_shape=jax.ShapeDtypeStruct((N // 8, 128), jnp.float32),
    )(a128, g2_128, dinv128, b2_128)


@jax.jit
def kernel(x, edge_index, W1, b1, W2, b2):
    f32 = jnp.float32
    ei_flat = edge_index.reshape(2 * E)
    zerosN = jnp.zeros((N,), f32)
    zeros = jnp.zeros((N, 16), f32)
    eye8 = jnp.eye(8, dtype=f32)
    W1bd = jnp.kron(eye8, W1)
    W2p = jnp.zeros((HID, 16), f32).at[:, :NCLS].set(W2)
    W2bd = jnp.kron(eye8, W2p)
    b1_128 = jnp.tile(b1, 8)[None, :]
    b2p = jnp.zeros((16,), f32).at[:NCLS].set(b2)
    b2_128 = jnp.tile(b2p, 8)[None, :]
    x_r = x.reshape(N // 8, 8 * F_IN)

    h1_128 = _tc1a_call(x_r, W1bd)
    dinv16 = _deg_call(ei_flat, zerosN)
    dinv128 = dinv16.reshape(N // 8, 128)

    g1_128 = _tc1b_call(h1_128, dinv128)
    g1 = g1_128.reshape(N, 16)

    agg1 = _agg_call(g1, ei_flat, zeros)
    a1_128 = agg1.reshape(2 * (N // 8), 128)
    g2_128 = _tc2_call(a1_128, g1_128, dinv128, b1_128, W2bd)
    g2 = g2_128.reshape(N, 16)

    agg2 = _agg_call(g2, ei_flat, zeros)
    a2_128 = agg2.reshape(2 * (N // 8), 128)
    o_128 = _tc3_call(a2_128, g2_128, dinv128, b2_128)
    return o_128.reshape(N, 16)[:, :NCLS]
